# baseline (speedup 1.0000x reference)
import functools

import jax
import jax.numpy as jnp
from jax import lax
from jax.experimental import pallas as pl
from jax.experimental.pallas import tpu as pltpu
from jax.experimental.pallas import tpu_sc as plsc

N = 10000
E = 320000
D = 128
G = 64
EPS = 1e-5

NTEC = 32
CHUNK = 128
CH = 80
E_PAD = NTEC * CH * CHUNK
N_PAD = 10240
ROWS_PER_TILE = N_PAD // 16
RSQ = 1.0 / (1.0 + EPS) ** 0.5


def _fill(ref, val):
    v = jnp.full((16,), val, jnp.float32)

    def fl(r, _):
        for cc in range(D // 16):
            ref[r, pl.ds(cc * 16, 16)] = v
        return 0

    lax.fori_loop(0, CHUNK, fl, 0)


def _make_agg(with_gather: bool):
    mesh = plsc.VectorSubcoreMesh(core_axis_name="c", subcore_axis_name="s")

    scratch = [
        pltpu.VMEM((CH, CHUNK), jnp.int32),
        pltpu.VMEM((CH, CHUNK), jnp.int32),
        pltpu.VMEM((CHUNK, D), jnp.float32),
        pltpu.VMEM_SHARED((N_PAD, D), jnp.float32),
        pltpu.SemaphoreType.DMA,
    ]

    def body(h_hbm, srcs_hbm, dsts_hbm, out_hbm, src_v, dst_v, rows_v,
             acc_sh, sem):
        c = lax.axis_index("c")
        s = lax.axis_index("s")
        w = s * 2 + c

        _fill(rows_v, 0.0)
        base = s * ROWS_PER_TILE
        for k in range(ROWS_PER_TILE // CHUNK):
            pltpu.sync_copy(rows_v, acc_sh.at[pl.ds(base + k * CHUNK, CHUNK)])
        plsc.subcore_barrier()

        if with_gather:
            pltpu.sync_copy(srcs_hbm.at[w], src_v)
        else:
            _fill(rows_v, 1.0)
        pltpu.sync_copy(dsts_hbm.at[w], dst_v)

        def step(j, _):
            if with_gather:
                pltpu.async_copy(h_hbm.at[src_v.at[j]], rows_v, sem).wait()
            pltpu.sync_copy(rows_v, acc_sh.at[dst_v.at[j]], add=True)
            return 0

        lax.fori_loop(0, CH, step, 0)
        plsc.subcore_barrier()

        for k in range(ROWS_PER_TILE // CHUNK):
            pltpu.sync_copy(
                acc_sh.at[pl.ds(base + k * CHUNK, CHUNK)],
                out_hbm.at[c, pl.ds(base + k * CHUNK, CHUNK)],
            )

    return pl.kernel(
        body,
        mesh=mesh,
        out_type=jax.ShapeDtypeStruct((2, N_PAD, D), jnp.float32),
        scratch_types=scratch,
    )


_agg = _make_agg(True)
_deg_call = _make_agg(False)


def _layer_body(p_ref, d_ref, h_ref, wl_ref, wr_ref, bl_ref, g_ref, bt_ref,
                o_ref):
    agg = p_ref[0] + p_ref[1]
    deg = d_ref[0, :, 0:1] + d_ref[1, :, 0:1]
    dinv = 1.0 / jnp.maximum(deg, 1.0)
    mean = agg * dinv
    y = (jnp.dot(mean, wl_ref[...], preferred_element_type=jnp.float32)
         + jnp.dot(h_ref[...], wr_ref[...], preferred_element_type=jnp.float32)
         + bl_ref[...])
    y = y * (g_ref[...] * RSQ) + bt_ref[...]
    o_ref[...] = jnp.maximum(y, 0.0)


BLK = 512

_layer_call = pl.pallas_call(
    _layer_body,
    grid=(N_PAD // BLK,),
    in_specs=[
        pl.BlockSpec((2, BLK, D), lambda i: (0, i, 0)),
        pl.BlockSpec((2, BLK, D), lambda i: (0, i, 0)),
        pl.BlockSpec((BLK, D), lambda i: (i, 0)),
        pl.BlockSpec((D, D), lambda i: (0, 0)),
        pl.BlockSpec((D, D), lambda i: (0, 0)),
        pl.BlockSpec((1, D), lambda i: (0, 0)),
        pl.BlockSpec((1, D), lambda i: (0, 0)),
        pl.BlockSpec((1, D), lambda i: (0, 0)),
    ],
    out_specs=pl.BlockSpec((BLK, D), lambda i: (i, 0)),
    out_shape=jax.ShapeDtypeStruct((N_PAD, D), jnp.float32),
)


def _pool_body(h_ref, b_ref, wp1_ref, bp1_ref, wp2_ref, bp2_ref, o_ref,
               pooled, cnt):
    i = pl.program_id(0)

    @pl.when(i == 0)
    def _():
        pooled[...] = jnp.zeros_like(pooled)
        cnt[...] = jnp.zeros_like(cnt)

    bvals = b_ref[...]
    gids = lax.broadcasted_iota(jnp.int32, (BLK, G), 1)
    oh = (bvals == gids).astype(jnp.float32)
    dn = (((0,), (0,)), ((), ()))
    pooled[...] += lax.dot_general(oh, h_ref[...], dn,
                                   preferred_element_type=jnp.float32)
    cnt[...] += lax.dot_general(oh, jnp.ones((BLK, D), jnp.float32), dn,
                                preferred_element_type=jnp.float32)

    @pl.when(i == pl.num_programs(0) - 1)
    def _():
        pm = pooled[...] / jnp.maximum(cnt[...], 1.0)
        t = jnp.maximum(
            jnp.dot(pm, wp1_ref[...], preferred_element_type=jnp.float32)
            + bp1_ref[...], 0.0)
        z = (jnp.dot(t, wp2_ref[...], preferred_element_type=jnp.float32)
             + bp2_ref[...])
        ss = jnp.sum(z * z, axis=1, keepdims=True)
        nrm = jnp.maximum(jnp.sqrt(ss), 1e-12)
        o_ref[...] = z / nrm


_pool_call = pl.pallas_call(
    _pool_body,
    grid=(N_PAD // BLK,),
    in_specs=[
        pl.BlockSpec((BLK, D), lambda i: (i, 0)),
        pl.BlockSpec((BLK, 1), lambda i: (i, 0)),
        pl.BlockSpec((D, D), lambda i: (0, 0)),
        pl.BlockSpec((1, D), lambda i: (0, 0)),
        pl.BlockSpec((D, D), lambda i: (0, 0)),
        pl.BlockSpec((1, D), lambda i: (0, 0)),
    ],
    out_specs=pl.BlockSpec((G, D), lambda i: (0, 0)),
    out_shape=jax.ShapeDtypeStruct((G, D), jnp.float32),
    scratch_shapes=[
        pltpu.VMEM((G, D), jnp.float32),
        pltpu.VMEM((G, D), jnp.float32),
    ],
)


def kernel(x, edge_index, batch, Wl_0, bl_0, Wr_0, g_0, bt_0, Wl_1, bl_1,
           Wr_1, g_1, bt_1, Wl_2, bl_2, Wr_2, g_2, bt_2, Wl_3, bl_3, Wr_3,
           g_3, bt_3, Wp1, bp1, Wp2, bp2):
    xp = jnp.pad(x, ((0, N_PAD - N), (0, 0)))
    src = jnp.concatenate(
        [edge_index[0], jnp.zeros((E_PAD - E,), jnp.int32)])
    pad_dst = N + jnp.arange(E_PAD - E, dtype=jnp.int32) % (N_PAD - N)
    dst = jnp.concatenate([edge_index[1], pad_dst])
    srcs = src.reshape(NTEC, CH, CHUNK)
    dsts = dst.reshape(NTEC, CH, CHUNK)
    bp = jnp.pad(batch, (0, N_PAD - N), constant_values=G).reshape(N_PAD, 1)

    layers = [
        (Wl_0, bl_0, Wr_0, g_0, bt_0),
        (Wl_1, bl_1, Wr_1, g_1, bt_1),
        (Wl_2, bl_2, Wr_2, g_2, bt_2),
        (Wl_3, bl_3, Wr_3, g_3, bt_3),
    ]

    h = xp
    deg = _deg_call(xp, srcs, dsts)
    for li, (Wl, bl, Wr, g, bt) in enumerate(layers):
        p = _agg(h, srcs, dsts)
        h = _layer_call(p, deg, h, Wl, Wr, bl.reshape(1, D),
                        g.reshape(1, D), bt.reshape(1, D))

    return _pool_call(h, bp, Wp1, bp1.reshape(1, D), Wp2,
                      bp2.reshape(1, 128))


# speedup vs baseline: 1.4372x; 1.4372x over previous
import functools

import jax
import jax.numpy as jnp
from jax import lax
from jax.experimental import pallas as pl
from jax.experimental.pallas import tpu as pltpu
from jax.experimental.pallas import tpu_sc as plsc

N = 10000
E = 320000
D = 128
G = 64
EPS = 1e-5

NTEC = 32
CHUNK = 128
CH = 79
E_PAD = NTEC * CH * CHUNK
N_PAD = 10240
ROWS_PER_TILE = N_PAD // 16
RSQ = 1.0 / (1.0 + EPS) ** 0.5


def _fill(ref, val):
    v = jnp.full((16,), val, jnp.float32)

    def fl(r, _):
        for cc in range(D // 16):
            ref[r, pl.ds(cc * 16, 16)] = v
        return 0

    lax.fori_loop(0, CHUNK, fl, 0)


def _make_agg(with_gather: bool):
    mesh = plsc.VectorSubcoreMesh(core_axis_name="c", subcore_axis_name="s")

    scratch = [
        pltpu.VMEM((CH, CHUNK), jnp.int32),
        pltpu.VMEM((CH, CHUNK), jnp.int32),
        pltpu.VMEM((CHUNK, D), jnp.float32),
        pltpu.VMEM_SHARED((N_PAD, D), jnp.float32),
        pltpu.SemaphoreType.DMA,
    ]

    def body(h_hbm, srcs_hbm, dsts_hbm, out_hbm, src_v, dst_v, rows_v,
             acc_sh, sem):
        c = lax.axis_index("c")
        s = lax.axis_index("s")
        w = s * 2 + c

        _fill(rows_v, 0.0)
        base = s * ROWS_PER_TILE
        for k in range(ROWS_PER_TILE // CHUNK):
            pltpu.sync_copy(rows_v, acc_sh.at[pl.ds(base + k * CHUNK, CHUNK)])
        plsc.subcore_barrier()

        if with_gather:
            pltpu.sync_copy(srcs_hbm.at[w], src_v)
        else:
            _fill(rows_v, 1.0)
        pltpu.sync_copy(dsts_hbm.at[w], dst_v)

        def step(j, _):
            if with_gather:
                pltpu.async_copy(h_hbm.at[src_v.at[j]], rows_v, sem).wait()
            pltpu.sync_copy(rows_v, acc_sh.at[dst_v.at[j]], add=True)
            return 0

        lax.fori_loop(0, CH, step, 0)
        plsc.subcore_barrier()

        for k in range(ROWS_PER_TILE // CHUNK):
            pltpu.sync_copy(
                acc_sh.at[pl.ds(base + k * CHUNK, CHUNK)],
                out_hbm.at[c, pl.ds(base + k * CHUNK, CHUNK)],
            )

    return pl.kernel(
        body,
        mesh=mesh,
        out_type=jax.ShapeDtypeStruct((2, N_PAD, D), jnp.float32),
        scratch_types=scratch,
    )


_agg = _make_agg(True)
_deg_call = _make_agg(False)


def _layer_body(p_ref, d_ref, h_ref, wl_ref, wr_ref, bl_ref, g_ref, bt_ref,
                o_ref):
    agg = p_ref[0] + p_ref[1]
    deg = d_ref[0, :, 0:1] + d_ref[1, :, 0:1]
    dinv = 1.0 / jnp.maximum(deg, 1.0)
    mean = agg * dinv
    y = (jnp.dot(mean, wl_ref[...], preferred_element_type=jnp.float32)
         + jnp.dot(h_ref[...], wr_ref[...], preferred_element_type=jnp.float32)
         + bl_ref[...])
    y = y * (g_ref[...] * RSQ) + bt_ref[...]
    o_ref[...] = jnp.maximum(y, 0.0)


BLK = 512

_layer_call = pl.pallas_call(
    _layer_body,
    grid=(N_PAD // BLK,),
    in_specs=[
        pl.BlockSpec((2, BLK, D), lambda i: (0, i, 0)),
        pl.BlockSpec((2, BLK, D), lambda i: (0, i, 0)),
        pl.BlockSpec((BLK, D), lambda i: (i, 0)),
        pl.BlockSpec((D, D), lambda i: (0, 0)),
        pl.BlockSpec((D, D), lambda i: (0, 0)),
        pl.BlockSpec((1, D), lambda i: (0, 0)),
        pl.BlockSpec((1, D), lambda i: (0, 0)),
        pl.BlockSpec((1, D), lambda i: (0, 0)),
    ],
    out_specs=pl.BlockSpec((BLK, D), lambda i: (i, 0)),
    out_shape=jax.ShapeDtypeStruct((N_PAD, D), jnp.float32),
)


def _pool_body(h_ref, b_ref, wp1_ref, bp1_ref, wp2_ref, bp2_ref, o_ref,
               pooled, cnt):
    i = pl.program_id(0)

    @pl.when(i == 0)
    def _():
        pooled[...] = jnp.zeros_like(pooled)
        cnt[...] = jnp.zeros_like(cnt)

    bvals = b_ref[...]
    gids = lax.broadcasted_iota(jnp.int32, (BLK, G), 1)
    oh = (bvals == gids).astype(jnp.float32)
    dn = (((0,), (0,)), ((), ()))
    pooled[...] += lax.dot_general(oh, h_ref[...], dn,
                                   preferred_element_type=jnp.float32)
    cnt[...] += lax.dot_general(oh, jnp.ones((BLK, D), jnp.float32), dn,
                                preferred_element_type=jnp.float32)

    @pl.when(i == pl.num_programs(0) - 1)
    def _():
        pm = pooled[...] / jnp.maximum(cnt[...], 1.0)
        t = jnp.maximum(
            jnp.dot(pm, wp1_ref[...], preferred_element_type=jnp.float32)
            + bp1_ref[...], 0.0)
        z = (jnp.dot(t, wp2_ref[...], preferred_element_type=jnp.float32)
             + bp2_ref[...])
        ss = jnp.sum(z * z, axis=1, keepdims=True)
        nrm = jnp.maximum(jnp.sqrt(ss), 1e-12)
        o_ref[...] = z / nrm


_pool_call = pl.pallas_call(
    _pool_body,
    grid=(N_PAD // BLK,),
    in_specs=[
        pl.BlockSpec((BLK, D), lambda i: (i, 0)),
        pl.BlockSpec((BLK, 1), lambda i: (i, 0)),
        pl.BlockSpec((D, D), lambda i: (0, 0)),
        pl.BlockSpec((1, D), lambda i: (0, 0)),
        pl.BlockSpec((D, D), lambda i: (0, 0)),
        pl.BlockSpec((1, D), lambda i: (0, 0)),
    ],
    out_specs=pl.BlockSpec((G, D), lambda i: (0, 0)),
    out_shape=jax.ShapeDtypeStruct((G, D), jnp.float32),
    scratch_shapes=[
        pltpu.VMEM((G, D), jnp.float32),
        pltpu.VMEM((G, D), jnp.float32),
    ],
)


def kernel(x, edge_index, batch, Wl_0, bl_0, Wr_0, g_0, bt_0, Wl_1, bl_1,
           Wr_1, g_1, bt_1, Wl_2, bl_2, Wr_2, g_2, bt_2, Wl_3, bl_3, Wr_3,
           g_3, bt_3, Wp1, bp1, Wp2, bp2):
    xp = jnp.pad(x, ((0, N_PAD - N), (0, 0)))
    src = jnp.concatenate(
        [edge_index[0], jnp.zeros((E_PAD - E,), jnp.int32)])
    dst = jnp.concatenate(
        [edge_index[1], jnp.full((E_PAD - E,), N, jnp.int32)])
    srcs = src.reshape(NTEC, CH, CHUNK)
    dsts = dst.reshape(NTEC, CH, CHUNK)
    bp = jnp.pad(batch, (0, N_PAD - N), constant_values=G).reshape(N_PAD, 1)

    layers = [
        (Wl_0, bl_0, Wr_0, g_0, bt_0),
        (Wl_1, bl_1, Wr_1, g_1, bt_1),
        (Wl_2, bl_2, Wr_2, g_2, bt_2),
        (Wl_3, bl_3, Wr_3, g_3, bt_3),
    ]

    h = xp
    deg = _deg_call(xp, srcs, dsts)
    for li, (Wl, bl, Wr, g, bt) in enumerate(layers):
        p = _agg(h, srcs, dsts)
        h = _layer_call(p, deg, h, Wl, Wr, bl.reshape(1, D),
                        g.reshape(1, D), bt.reshape(1, D))

    return _pool_call(h, bp, Wp1, bp1.reshape(1, D), Wp2,
                      bp2.reshape(1, 128))
